# baseline (speedup 1.0000x reference)
import functools

import jax
import jax.numpy as jnp
from jax import lax
from jax.experimental import pallas as pl
from jax.experimental.pallas import tpu as pltpu
from jax.experimental.pallas import tpu_sc as plsc

P = (1 << 31) - 1
B_ROWS = 500000
W_SIZE = 32000000
DIM = 64
N_HASH = 4
N_IDX = 425984
NC = 2
NS = 16
LANES = 16
NW = NC * NS
B_PER_W = N_IDX // NW
C = 64
NCHUNK = B_PER_W // C
PD = 4

_U = jnp.uint32


def _hash_mod(xl, xh, a_lo, a_hi, b, m, inv_m):
    t3 = a_hi * xh
    t2 = a_hi * xl
    t1 = a_lo * xh
    t0 = a_lo * xl
    pu = _U(P)
    r3 = ((t3 & _U(31)) << _U(26)) + (t3 >> _U(5))
    r2 = ((t2 & _U(0x7FFF)) << _U(16)) + (t2 >> _U(15))
    r1 = ((t1 & _U(0x1FFFFF)) << _U(10)) + (t1 >> _U(21))
    s = r3 + r2
    s = (s & pu) + (s >> _U(31))
    s = s + r1
    s = (s & pu) + (s >> _U(31))
    s = s + t0
    s = (s & pu) + (s >> _U(31))
    s = s + b
    s = (s & pu) + (s >> _U(31))
    s = jnp.where(s >= pu, s - pu, s)
    q = (s.astype(jnp.float32) * inv_m).astype(jnp.int32).astype(_U)
    rem = s - q * _U(m)
    rem = jnp.where(rem >= _U(1 << 31), rem + _U(m), rem)
    rem = jnp.where(rem >= _U(m), rem - _U(m), rem)
    return rem.astype(jnp.int32)


def _body(consts_hbm, x_hbm, table_hbm, weights_hbm, out_hbm,
          consts_v, x_v, idx_v, rows_v, w_v, out_v, sems, wsems):
    wid = lax.axis_index("s") * jnp.int32(NC) + lax.axis_index("c")
    base = wid * jnp.int32(B_PER_W)
    pltpu.sync_copy(consts_hbm, consts_v)
    pltpu.sync_copy(x_hbm.at[pl.ds(base, B_PER_W)], x_v)

    def fire(g):
        pu = lax.rem(g, jnp.int32(PD))
        cbase = g * jnp.int32(C)

        def hash_grp(i, carry):
            sl = pl.ds(cbase + i * jnp.int32(LANES), LANES)
            xv = x_v[sl].astype(_U)
            xl = xv & _U(1023)
            xh = xv >> _U(10)
            osl = pl.ds(i * jnp.int32(LANES), LANES)
            for j in range(2 * N_HASH):
                a_lo = consts_v[3 * j + 0, :].astype(_U)
                a_hi = consts_v[3 * j + 1, :].astype(_U)
                b = consts_v[3 * j + 2, :].astype(_U)
                if j < N_HASH:
                    m, inv_m = B_ROWS, jnp.float32(1.0 / B_ROWS)
                else:
                    m, inv_m = W_SIZE, jnp.float32(1.0 / W_SIZE)
                idx_v[pu, j, osl] = _hash_mod(xl, xh, a_lo, a_hi, b,
                                              m, inv_m)
            return carry

        lax.fori_loop(0, C // LANES, hash_grp, jnp.int32(0))
        for h in range(N_HASH):
            pltpu.async_copy(
                table_hbm.at[idx_v.at[pu, jnp.int32(h)]],
                rows_v.at[pu, jnp.int32(h)], sems.at[pu])
        for h in range(N_HASH):
            pltpu.async_copy(
                weights_hbm.at[idx_v.at[pu, jnp.int32(N_HASH + h)]],
                w_v.at[pu, jnp.int32(h)], sems.at[pu])

    def proc(g):
        start = base + g * jnp.int32(C)
        pu = lax.rem(g, jnp.int32(PD))
        po = lax.rem(g, jnp.int32(2))
        for h in range(N_HASH):
            pltpu.make_async_copy(
                table_hbm.at[idx_v.at[pu, jnp.int32(h)]],
                rows_v.at[pu, jnp.int32(h)], sems.at[pu]).wait()
        for h in range(N_HASH):
            pltpu.make_async_copy(
                weights_hbm.at[idx_v.at[pu, jnp.int32(N_HASH + h)]],
                w_v.at[pu, jnp.int32(h)], sems.at[pu]).wait()

        @pl.when(g >= jnp.int32(2))
        def _():
            pltpu.make_async_copy(
                out_v.at[po], out_hbm.at[pl.ds(base, C)],
                wsems.at[po]).wait()

        def cmp_grp(gi, carry):
            nb = gi * jnp.int32(LANES)
            wv = [w_v[pu, jnp.int32(h), pl.ds(nb, LANES)]
                  for h in range(N_HASH)]
            for j in range(LANES):
                n = nb + jnp.int32(j)
                for dv in range(DIM // LANES):
                    dsl = pl.ds(dv * LANES, LANES)
                    acc = rows_v[pu, 0, n, dsl] * wv[0][j]
                    acc = acc + rows_v[pu, 1, n, dsl] * wv[1][j]
                    acc = acc + rows_v[pu, 2, n, dsl] * wv[2][j]
                    acc = acc + rows_v[pu, 3, n, dsl] * wv[3][j]
                    out_v[po, n, dsl] = acc
            return carry

        lax.fori_loop(0, C // LANES, cmp_grp, jnp.int32(0))
        pltpu.async_copy(out_v.at[po], out_hbm.at[pl.ds(start, C)],
                         wsems.at[po])

    def pro(i, carry):
        fire(i)
        return carry

    lax.fori_loop(0, PD, pro, jnp.int32(0))

    def step(g, carry):
        proc(g)
        fire(g + jnp.int32(PD))
        return carry

    lax.fori_loop(0, NCHUNK - PD, step, jnp.int32(0))

    def epi(i, carry):
        proc(jnp.int32(NCHUNK - PD) + i)
        return carry

    lax.fori_loop(0, PD, epi, jnp.int32(0))
    for po in range(2):
        pltpu.make_async_copy(
            out_v.at[jnp.int32(po)], out_hbm.at[pl.ds(base, C)],
            wsems.at[jnp.int32(po)]).wait()


@jax.jit
def _run(consts, x_i32, table, weights):
    mesh = plsc.VectorSubcoreMesh(
        core_axis_name="c", subcore_axis_name="s",
        num_cores=NC, num_subcores=NS)
    f = pl.kernel(
        _body,
        out_type=jax.ShapeDtypeStruct((N_IDX, DIM), jnp.float32),
        mesh=mesh,
        scratch_types=[
            pltpu.VMEM((3 * 2 * N_HASH, LANES), jnp.int32),
            pltpu.VMEM((B_PER_W,), jnp.int32),
            pltpu.VMEM((PD, 2 * N_HASH, C), jnp.int32),
            pltpu.VMEM((PD, N_HASH, C, DIM), jnp.float32),
            pltpu.VMEM((PD, N_HASH, C), jnp.float32),
            pltpu.VMEM((2, C, DIM), jnp.float32),
            pltpu.SemaphoreType.DMA((PD,)),
            pltpu.SemaphoreType.DMA((2,)),
        ],
        compiler_params=pltpu.CompilerParams(use_tc_tiling_on_sc=False),
    )
    return f(consts, x_i32, table, weights)


def kernel(x, table, weights, a0, b0, a1, b1):
    a_all = jnp.concatenate([a0, a1])
    b_all = jnp.concatenate([b0, b1])
    a_lo = (a_all & 0xFFFF).astype(jnp.int32)
    a_hi = (a_all >> 16).astype(jnp.int32)
    b32 = b_all.astype(jnp.int32)
    trip = jnp.stack([a_lo, a_hi, b32], axis=1).reshape(3 * 2 * N_HASH)
    consts = jnp.tile(trip[:, None], (1, LANES)).astype(jnp.int32)
    x_i32 = x.astype(jnp.int32)
    with jax.enable_x64(False):
        return _run(consts, x_i32, table, weights)


# speedup vs baseline: 1.2392x; 1.2392x over previous
import functools

import jax
import jax.numpy as jnp
from jax import lax
from jax.experimental import pallas as pl
from jax.experimental.pallas import tpu as pltpu
from jax.experimental.pallas import tpu_sc as plsc

P = (1 << 31) - 1
B_ROWS = 500000
W_SIZE = 32000000
DIM = 64
N_HASH = 4
N_IDX = 425984
NC = 2
NS = 16
LANES = 16
NW = NC * NS
B_PER_W = N_IDX // NW
C = 128
NCHUNK = B_PER_W // C

_U = jnp.uint32


def _hash_mod(xl, xh, a_lo, a_hi, b, m, inv_m):
    t3 = a_hi * xh
    t2 = a_hi * xl
    t1 = a_lo * xh
    t0 = a_lo * xl
    pu = _U(P)
    r3 = ((t3 & _U(31)) << _U(26)) + (t3 >> _U(5))
    r2 = ((t2 & _U(0x7FFF)) << _U(16)) + (t2 >> _U(15))
    r1 = ((t1 & _U(0x1FFFFF)) << _U(10)) + (t1 >> _U(21))
    s = r3 + r2
    s = (s & pu) + (s >> _U(31))
    s = s + r1
    s = (s & pu) + (s >> _U(31))
    s = s + t0
    s = (s & pu) + (s >> _U(31))
    s = s + b
    s = (s & pu) + (s >> _U(31))
    s = jnp.where(s >= pu, s - pu, s)
    q = (s.astype(jnp.float32) * inv_m).astype(jnp.int32).astype(_U)
    rem = s - q * _U(m)
    rem = jnp.where(rem >= _U(1 << 31), rem + _U(m), rem)
    rem = jnp.where(rem >= _U(m), rem - _U(m), rem)
    return rem.astype(jnp.int32)


def _body(consts_hbm, x_hbm, table_hbm, weights_hbm, out_hbm,
          consts_v, x_v, idx_v, rows_v, w_v, out_v,
          sem0, sem1, wsem0, wsem1):
    wid = lax.axis_index("s") * jnp.int32(NC) + lax.axis_index("c")
    base = wid * jnp.int32(B_PER_W)
    pltpu.sync_copy(consts_hbm, consts_v)
    pltpu.sync_copy(x_hbm.at[pl.ds(base, B_PER_W)], x_v)
    sems = (sem0, sem1)
    wsems = (wsem0, wsem1)

    def fire(g, p):
        pu = jnp.int32(p)
        cbase = g * jnp.int32(C)

        def hash_grp(i, carry):
            sl = pl.ds(cbase + i * jnp.int32(LANES), LANES)
            xv = x_v[sl].astype(_U)
            xl = xv & _U(1023)
            xh = xv >> _U(10)
            osl = pl.ds(i * jnp.int32(LANES), LANES)
            for j in range(2 * N_HASH):
                a_lo = consts_v[3 * j + 0, :].astype(_U)
                a_hi = consts_v[3 * j + 1, :].astype(_U)
                b = consts_v[3 * j + 2, :].astype(_U)
                if j < N_HASH:
                    m, inv_m = B_ROWS, jnp.float32(1.0 / B_ROWS)
                else:
                    m, inv_m = W_SIZE, jnp.float32(1.0 / W_SIZE)
                idx_v[pu, j, osl] = _hash_mod(xl, xh, a_lo, a_hi, b,
                                              m, inv_m)
            return carry

        lax.fori_loop(jnp.int32(0), jnp.int32(C // LANES), hash_grp,
                      jnp.int32(0))
        for h in range(N_HASH):
            pltpu.async_copy(
                table_hbm.at[idx_v.at[pu, jnp.int32(h)]],
                rows_v.at[pu, jnp.int32(h)], sems[p])
        for h in range(N_HASH):
            pltpu.async_copy(
                weights_hbm.at[idx_v.at[pu, jnp.int32(N_HASH + h)]],
                w_v.at[pu, jnp.int32(h)], sems[p])

    def drain_wb(p):
        pltpu.make_async_copy(
            out_v.at[jnp.int32(p)], out_hbm.at[pl.ds(base, C)],
            wsems[p]).wait()

    def proc(g, p, wb_pending):
        start = base + g * jnp.int32(C)
        pu = jnp.int32(p)
        for h in range(N_HASH):
            pltpu.make_async_copy(
                table_hbm.at[idx_v.at[pu, jnp.int32(h)]],
                rows_v.at[pu, jnp.int32(h)], sems[p]).wait()
        for h in range(N_HASH):
            pltpu.make_async_copy(
                weights_hbm.at[idx_v.at[pu, jnp.int32(N_HASH + h)]],
                w_v.at[pu, jnp.int32(h)], sems[p]).wait()
        if wb_pending:
            drain_wb(p)

        def cmp_grp(gi, carry):
            nb = gi * jnp.int32(LANES)
            wv = [w_v[pu, jnp.int32(h), pl.ds(nb, LANES)]
                  for h in range(N_HASH)]
            for j in range(LANES):
                n = nb + jnp.int32(j)
                for dv in range(DIM // LANES):
                    dsl = pl.ds(dv * LANES, LANES)
                    acc = rows_v[pu, 0, n, dsl] * wv[0][j]
                    acc = acc + rows_v[pu, 1, n, dsl] * wv[1][j]
                    acc = acc + rows_v[pu, 2, n, dsl] * wv[2][j]
                    acc = acc + rows_v[pu, 3, n, dsl] * wv[3][j]
                    out_v[pu, n, dsl] = acc
            return carry

        lax.fori_loop(jnp.int32(0), jnp.int32(C // LANES), cmp_grp,
                      jnp.int32(0))
        pltpu.async_copy(out_v.at[pu], out_hbm.at[pl.ds(start, C)],
                         wsems[p])

    fire(jnp.int32(0), 0)
    fire(jnp.int32(1), 1)
    proc(jnp.int32(0), 0, False)
    fire(jnp.int32(2), 0)
    proc(jnp.int32(1), 1, False)
    fire(jnp.int32(3), 1)

    def pair_body(i, carry):
        g = jnp.int32(2) + i * jnp.int32(2)
        proc(g, 0, True)
        fire(g + jnp.int32(2), 0)
        proc(g + jnp.int32(1), 1, True)
        fire(g + jnp.int32(3), 1)
        return carry

    lax.fori_loop(jnp.int32(0), jnp.int32((NCHUNK - 4) // 2), pair_body,
                  jnp.int32(0))

    gl = jnp.int32(NCHUNK - 2)
    proc(gl, 0, True)
    proc(gl + jnp.int32(1), 1, True)
    drain_wb(0)
    drain_wb(1)


@jax.jit
def _run(consts, x_i32, table, weights):
    mesh = plsc.VectorSubcoreMesh(
        core_axis_name="c", subcore_axis_name="s",
        num_cores=NC, num_subcores=NS)
    f = pl.kernel(
        _body,
        out_type=jax.ShapeDtypeStruct((N_IDX, DIM), jnp.float32),
        mesh=mesh,
        scratch_types=[
            pltpu.VMEM((3 * 2 * N_HASH, LANES), jnp.int32),
            pltpu.VMEM((B_PER_W,), jnp.int32),
            pltpu.VMEM((2, 2 * N_HASH, C), jnp.int32),
            pltpu.VMEM((2, N_HASH, C, DIM), jnp.float32),
            pltpu.VMEM((2, N_HASH, C), jnp.float32),
            pltpu.VMEM((2, C, DIM), jnp.float32),
            pltpu.SemaphoreType.DMA,
            pltpu.SemaphoreType.DMA,
            pltpu.SemaphoreType.DMA,
            pltpu.SemaphoreType.DMA,
        ],
        compiler_params=pltpu.CompilerParams(use_tc_tiling_on_sc=False),
    )
    return f(consts, x_i32, table, weights)


def kernel(x, table, weights, a0, b0, a1, b1):
    a_all = jnp.concatenate([a0, a1])
    b_all = jnp.concatenate([b0, b1])
    a_lo = (a_all & 0xFFFF).astype(jnp.int32)
    a_hi = (a_all >> 16).astype(jnp.int32)
    b32 = b_all.astype(jnp.int32)
    trip = jnp.stack([a_lo, a_hi, b32], axis=1).reshape(3 * 2 * N_HASH)
    consts = jnp.tile(trip[:, None], (1, LANES)).astype(jnp.int32)
    x_i32 = x.astype(jnp.int32)
    return _run(consts, x_i32, table, weights)
